# speedup vs baseline: 1.0113x; 1.0113x over previous
import jax
import jax.numpy as jnp
from jax.experimental import pallas as pl


def _add_kernel(p_ref, e_ref, o_ref):
    o_ref[...] = p_ref[...] + e_ref[...]


def kernel(patch, position_embedding):
    B, S, D = patch.shape
    pos = position_embedding[:S]
    return pl.pallas_call(
        _add_kernel,
        grid=(B,),
        in_specs=[
            pl.BlockSpec((1, S, D), lambda b: (b, 0, 0)),
            pl.BlockSpec((S, D), lambda b: (0, 0)),
        ],
        out_specs=pl.BlockSpec((1, S, D), lambda b: (b, 0, 0)),
        out_shape=jax.ShapeDtypeStruct((B, S, D), patch.dtype),
    )(patch, pos)
